# Initial kernel scaffold; baseline (speedup 1.0000x reference)
import jax
import jax.numpy as jnp
from jax.experimental import pallas as pl


def kernel(feat, edge_index, edge_feats, W_self, W_neigh, bias):
    raise NotImplementedError("write your pallas kernel here")


# speedup vs baseline: 5.6568x; 5.6568x over previous
import functools

import jax
import jax.numpy as jnp
from jax import lax
from jax.experimental import pallas as pl
from jax.experimental.pallas import tpu as pltpu
from jax.experimental.pallas import tpu_sc as plsc

_NC = 2
_NS = 16
_NW = _NC * _NS
_C = 80


def _npad(n):
  return ((n + _C * _NS - 1) // (_C * _NS)) * (_C * _NS)


def _mesh():
  return plsc.VectorSubcoreMesh(core_axis_name="c", subcore_axis_name="s")


def _sc_feat_sums(feat, eidx):
  n, d = feat.shape
  _, nw, nch, c = eidx.shape
  npad = _npad(n)
  rows_per_tile = npad // _NS

  @functools.partial(
      pl.kernel,
      out_type=jax.ShapeDtypeStruct((_NC, npad, d), jnp.float32),
      mesh=_mesh(),
      scratch_types=[
          pltpu.VMEM((nch, c), jnp.int32),
          pltpu.VMEM((nch, c), jnp.int32),
          pltpu.VMEM((c, d), jnp.float32),
          pltpu.VMEM_SHARED((npad, d), jnp.float32),
          pltpu.SemaphoreType.DMA,
      ],
  )
  def k(eidx_h, feat_h, psum_h, sidx, didx, rows, acc, sem):
    cid = lax.axis_index("c")
    sid = lax.axis_index("s")
    wid = sid * _NC + cid

    zeros16 = jnp.zeros((16,), jnp.float32)

    def zrow(i, carry):
      for j in range(d // 16):
        rows[i, pl.ds(j * 16, 16)] = zeros16
      return carry
    lax.fori_loop(0, c, zrow, 0)

    r0 = sid * rows_per_tile
    for kk in range(rows_per_tile // c):
      pltpu.sync_copy(rows, acc.at[pl.ds(r0 + kk * c, c)])

    pltpu.sync_copy(eidx_h.at[0, wid], sidx)
    pltpu.sync_copy(eidx_h.at[1, wid], didx)
    plsc.subcore_barrier()

    def chunk(i, carry):
      pltpu.async_copy(feat_h.at[sidx.at[i]], rows, sem).wait()
      pltpu.sync_copy(rows, acc.at[didx.at[i]], add=True)
      return carry
    lax.fori_loop(0, nch, chunk, 0)

    plsc.subcore_barrier()
    pltpu.sync_copy(acc.at[pl.ds(r0, rows_per_tile)],
                    psum_h.at[cid, pl.ds(r0, rows_per_tile)])

  return k(eidx, feat)


def _sc_ef_deg_sums(ef, eidx, n):
  _, nw, nch, c = eidx.shape
  de = ef.shape[-1]
  npad = _npad(n)
  rows_per_tile = npad // _NS

  @functools.partial(
      pl.kernel,
      out_type=[
          jax.ShapeDtypeStruct((_NC, npad, de), jnp.float32),
          jax.ShapeDtypeStruct((_NC, npad, de), jnp.float32),
      ],
      mesh=_mesh(),
      scratch_types=[
          pltpu.VMEM((nch, c), jnp.int32),
          pltpu.VMEM((c, de), jnp.float32),
          pltpu.VMEM((c, de), jnp.float32),
          pltpu.VMEM_SHARED((npad, de), jnp.float32),
          pltpu.VMEM_SHARED((npad, de), jnp.float32),
      ],
      compiler_params=pltpu.CompilerParams(use_tc_tiling_on_sc=False),
  )
  def k(eidx_h, ef_h, pef_h, pdeg_h, didx, efb, onesb, eacc, dacc):
    cid = lax.axis_index("c")
    sid = lax.axis_index("s")
    wid = sid * _NC + cid

    zeros16 = jnp.zeros((16,), jnp.float32)
    ones16 = jnp.ones((16,), jnp.float32)

    def zrow(i, carry):
      efb[i, :] = zeros16
      onesb[i, :] = ones16
      return carry
    lax.fori_loop(0, c, zrow, 0)

    r0 = sid * rows_per_tile
    for kk in range(rows_per_tile // c):
      pltpu.sync_copy(efb, eacc.at[pl.ds(r0 + kk * c, c)])
      pltpu.sync_copy(efb, dacc.at[pl.ds(r0 + kk * c, c)])

    pltpu.sync_copy(eidx_h.at[1, wid], didx)
    plsc.subcore_barrier()

    def chunk(i, carry):
      pltpu.sync_copy(ef_h.at[wid, i], efb)
      pltpu.sync_copy(efb, eacc.at[didx.at[i]], add=True)
      pltpu.sync_copy(onesb, dacc.at[didx.at[i]], add=True)
      return carry
    lax.fori_loop(0, nch, chunk, 0)

    plsc.subcore_barrier()
    pltpu.sync_copy(eacc.at[pl.ds(r0, rows_per_tile)],
                    pef_h.at[cid, pl.ds(r0, rows_per_tile)])
    pltpu.sync_copy(dacc.at[pl.ds(r0, rows_per_tile)],
                    pdeg_h.at[cid, pl.ds(r0, rows_per_tile)])

  return k(eidx, ef)


def _combine(feat, psum, pef, pdeg, w_self, w_neigh, bias2d):
  n, d = feat.shape
  de = pef.shape[-1]
  dk = w_neigh.shape[0]
  blk = 1000
  grid = (n // blk,)

  def body(f_ref, ps_ref, pe_ref, pd_ref, ws_ref, wn_ref, b_ref, o_ref):
    s = ps_ref[0] + ps_ref[1]
    e = pe_ref[0] + pe_ref[1]
    dg = pd_ref[0] + pd_ref[1]
    scale = 1.0 / jnp.maximum(dg[:, 0:1], 1.0)
    wn = wn_ref[...]
    h = (jnp.dot(s, wn[0:d], preferred_element_type=jnp.float32)
         + jnp.dot(e, wn[d:dk], preferred_element_type=jnp.float32))
    o_ref[...] = (jnp.dot(f_ref[...], ws_ref[...],
                          preferred_element_type=jnp.float32)
                  + h * scale + b_ref[...])

  return pl.pallas_call(
      body,
      grid=grid,
      in_specs=[
          pl.BlockSpec((blk, d), lambda i: (i, 0)),
          pl.BlockSpec((_NC, blk, d), lambda i: (0, i, 0)),
          pl.BlockSpec((_NC, blk, de), lambda i: (0, i, 0)),
          pl.BlockSpec((_NC, blk, de), lambda i: (0, i, 0)),
          pl.BlockSpec((d, d), lambda i: (0, 0)),
          pl.BlockSpec((dk, d), lambda i: (0, 0)),
          pl.BlockSpec((1, d), lambda i: (0, 0)),
      ],
      out_specs=pl.BlockSpec((blk, d), lambda i: (i, 0)),
      out_shape=jax.ShapeDtypeStruct((n, d), jnp.float32),
  )(feat, psum, pef, pdeg, w_self, w_neigh, bias2d)


def kernel(feat, edge_index, edge_feats, W_self, W_neigh, bias):
  n, d = feat.shape
  e_total = edge_index.shape[1]
  de = edge_feats.shape[1]
  epw = e_total // _NW
  nch = epw // _C
  eidx = edge_index.reshape(2, _NW, nch, _C)
  ef = edge_feats.reshape(_NW, nch, _C, de)
  psum = _sc_feat_sums(feat, eidx)
  pef, pdeg = _sc_ef_deg_sums(ef, eidx, n)
  return _combine(feat, psum, pef, pdeg, W_self, W_neigh,
                  bias.reshape(1, d))


# speedup vs baseline: 7.0062x; 1.2385x over previous
import functools

import jax
import jax.numpy as jnp
from jax import lax
from jax.experimental import pallas as pl
from jax.experimental.pallas import tpu as pltpu
from jax.experimental.pallas import tpu_sc as plsc

_NC = 2
_NS = 16
_C = 80
_AUG = 32


def _sc_segment_sums(feat2, srcx, dstx, ef, n):
  dh = feat2.shape[-1]
  _, ns, nch, c = srcx.shape
  de = ef.shape[-1]
  npad = ((n + _C * _NS - 1) // (_C * _NS)) * (_C * _NS)
  rpt = npad // _NS
  nef = nch // _NC

  mesh = plsc.VectorSubcoreMesh(core_axis_name="c", subcore_axis_name="s")

  @functools.partial(
      pl.kernel,
      out_type=[
          jax.ShapeDtypeStruct((_NC, npad, dh), jnp.float32),
          jax.ShapeDtypeStruct((_NC, npad, _AUG), jnp.float32),
      ],
      mesh=mesh,
      scratch_types=[
          pltpu.VMEM((nch, c), jnp.int32),
          pltpu.VMEM((nch, c), jnp.int32),
          pltpu.VMEM((c, dh), jnp.float32),
          pltpu.VMEM((c, dh), jnp.float32),
          pltpu.VMEM((c, _AUG), jnp.float32),
          pltpu.VMEM((c, _AUG), jnp.float32),
          pltpu.VMEM((c, dh), jnp.float32),
          pltpu.VMEM((c, _AUG), jnp.float32),
          pltpu.VMEM_SHARED((npad, dh), jnp.float32),
          pltpu.VMEM_SHARED((npad, _AUG), jnp.float32),
          pltpu.SemaphoreType.DMA,
          pltpu.SemaphoreType.DMA,
          pltpu.SemaphoreType.DMA,
          pltpu.SemaphoreType.DMA,
      ],
      compiler_params=pltpu.CompilerParams(use_tc_tiling_on_sc=False),
  )
  def k(srcx_h, dstx_h, ef_h, feat2_h, psum_h, paug_h,
        sidx, didx, r0b, r1b, a0b, a1b, zb, zab, acc, eacc,
        g0, g1, e0, e1):
    cid = lax.axis_index("c")
    sid = lax.axis_index("s")

    pltpu.sync_copy(srcx_h.at[cid, sid], sidx)
    pltpu.sync_copy(dstx_h.at[sid], didx)

    pltpu.async_copy(feat2_h.at[sidx.at[0]], r0b, g0)
    pltpu.async_copy(feat2_h.at[sidx.at[1]], r1b, g1)

    zeros16 = jnp.zeros((16,), jnp.float32)
    onecol = jnp.where(lax.iota(jnp.int32, 16) == 0,
                       jnp.float32(1.0), jnp.float32(0.0))

    def zrow(i, carry):
      for j in range(dh // 16):
        zb[i, pl.ds(j * 16, 16)] = zeros16
      for j in range(_AUG // 16):
        zab[i, pl.ds(j * 16, 16)] = zeros16
      a0b[i, pl.ds(de, 16)] = onecol
      a1b[i, pl.ds(de, 16)] = onecol
      return carry
    lax.fori_loop(0, c, zrow, 0)

    rr0 = sid * rpt
    for kk in range(rpt // c):
      pltpu.sync_copy(zb, acc.at[pl.ds(rr0 + kk * c, c)])
      pltpu.sync_copy(zab, eacc.at[pl.ds(rr0 + kk * c, c)])
    plsc.subcore_barrier()

    eoff = cid * nef
    pltpu.async_copy(ef_h.at[sid, eoff], a0b.at[:, pl.ds(0, de)], e0)
    pltpu.async_copy(ef_h.at[sid, eoff + 1], a1b.at[:, pl.ds(0, de)], e1)

    def gwait(buf, sem, ci):
      pltpu.make_async_copy(feat2_h.at[sidx.at[ci]], buf, sem).wait()

    def ewait(buf, sem, ei):
      pltpu.make_async_copy(ef_h.at[sid, ei], buf.at[:, pl.ds(0, de)],
                            sem).wait()

    def body(g, carry):
      for (buf, sem, off) in ((r0b, g0, 0), (r1b, g1, 1),
                              (r0b, g0, 2), (r1b, g1, 3)):
        ci = 4 * g + off
        gwait(buf, sem, ci)
        pltpu.sync_copy(buf, acc.at[didx.at[ci]], add=True)
        pltpu.async_copy(feat2_h.at[sidx.at[ci + 2]], buf, sem)
      for (buf, sem, off) in ((a0b, e0, 0), (a1b, e1, 1)):
        ei = eoff + 2 * g + off
        ewait(buf, sem, ei)
        pltpu.sync_copy(buf, eacc.at[didx.at[ei]], add=True)
        nxt = 2 * g + off + 2

        @pl.when(nxt < nef)
        def _():
          pltpu.async_copy(ef_h.at[sid, eoff + nxt],
                           buf.at[:, pl.ds(0, de)], sem)
      return carry

    niter = (nch - 2) // 4
    lax.fori_loop(0, niter, body, 0)

    gwait(r0b, g0, nch - 2)
    pltpu.sync_copy(r0b, acc.at[didx.at[nch - 2]], add=True)
    gwait(r1b, g1, nch - 1)
    pltpu.sync_copy(r1b, acc.at[didx.at[nch - 1]], add=True)
    if nef % 2:
      ewait(a0b, e0, eoff + nef - 1)
      pltpu.sync_copy(a0b, eacc.at[didx.at[eoff + nef - 1]], add=True)

    plsc.subcore_barrier()
    pltpu.sync_copy(acc.at[pl.ds(rr0, rpt)],
                    psum_h.at[cid, pl.ds(rr0, rpt)])
    pltpu.sync_copy(eacc.at[pl.ds(rr0, rpt)],
                    paug_h.at[cid, pl.ds(rr0, rpt)])

  return k(srcx, dstx, ef, feat2)


def _combine(feat, psum, paug, w_self, w_neigh, bias2d):
  n, d = feat.shape
  dh = psum.shape[-1]
  dk = w_neigh.shape[0]
  de = dk - d
  blk = 1000
  grid = (n // blk,)

  def body(f_ref, ps_ref, pa_ref, ws_ref, wn_ref, b_ref, o_ref):
    a = pa_ref[0] + pa_ref[1]
    e = a[:, 0:de]
    dg = a[:, de:de + 1]
    scale = 1.0 / jnp.maximum(dg, 1.0)
    wn = wn_ref[...]
    h = (jnp.dot(ps_ref[0], wn[0:dh], preferred_element_type=jnp.float32)
         + jnp.dot(ps_ref[1], wn[dh:2 * dh],
                   preferred_element_type=jnp.float32)
         + jnp.dot(e, wn[2 * dh:dk], preferred_element_type=jnp.float32))
    o_ref[...] = (jnp.dot(f_ref[...], ws_ref[...],
                          preferred_element_type=jnp.float32)
                  + h * scale + b_ref[...])

  return pl.pallas_call(
      body,
      grid=grid,
      in_specs=[
          pl.BlockSpec((blk, d), lambda i: (i, 0)),
          pl.BlockSpec((_NC, blk, dh), lambda i: (0, i, 0)),
          pl.BlockSpec((_NC, blk, _AUG), lambda i: (0, i, 0)),
          pl.BlockSpec((d, d), lambda i: (0, 0)),
          pl.BlockSpec((dk, d), lambda i: (0, 0)),
          pl.BlockSpec((1, d), lambda i: (0, 0)),
      ],
      out_specs=pl.BlockSpec((blk, d), lambda i: (i, 0)),
      out_shape=jax.ShapeDtypeStruct((n, d), jnp.float32),
  )(feat, psum, paug, w_self, w_neigh, bias2d)


def kernel(feat, edge_index, edge_feats, W_self, W_neigh, bias):
  n, d = feat.shape
  e_total = edge_index.shape[1]
  de = edge_feats.shape[1]
  dh = d // _NC
  ept = e_total // _NS
  nch = ept // _C
  src = edge_index[0]
  dst = edge_index[1]
  feat2 = jnp.concatenate([feat[:, :dh], feat[:, dh:]], axis=0)
  srcx = jnp.stack([src, src + n]).reshape(2, _NS, nch, _C)
  dstx = dst.reshape(_NS, nch, _C)
  ef = edge_feats.reshape(_NS, nch, _C, de)
  psum, paug = _sc_segment_sums(feat2, srcx, dstx, ef, n)
  return _combine(feat, psum, paug, W_self, W_neigh, bias.reshape(1, d))


# speedup vs baseline: 7.5463x; 1.0771x over previous
import functools

import jax
import jax.numpy as jnp
from jax import lax
from jax.experimental import pallas as pl
from jax.experimental.pallas import tpu as pltpu
from jax.experimental.pallas import tpu_sc as plsc

_NC = 2
_NS = 16
_C = 80
_AUG = 32


def _sc_segment_sums(feat2, srcx, dst, ef, n):
  dh = feat2.shape[-1]
  e_total = dst.shape[0]
  de = ef.shape[-1]
  ept = e_total // _NS
  nch = ept // _C
  npad = ((n + _C * _NS - 1) // (_C * _NS)) * (_C * _NS)
  rpt = npad // _NS
  nef = nch // _NC

  mesh = plsc.VectorSubcoreMesh(core_axis_name="c", subcore_axis_name="s")

  @functools.partial(
      pl.kernel,
      out_type=[
          jax.ShapeDtypeStruct((_NC, npad, dh), jnp.float32),
          jax.ShapeDtypeStruct((_NC, npad, _AUG), jnp.float32),
      ],
      mesh=mesh,
      scratch_types=[
          pltpu.VMEM((ept,), jnp.int32),
          pltpu.VMEM((ept,), jnp.int32),
          pltpu.VMEM((_C, dh), jnp.float32),
          pltpu.VMEM((_C, dh), jnp.float32),
          pltpu.VMEM((_C, _AUG), jnp.float32),
          pltpu.VMEM((_C, _AUG), jnp.float32),
          pltpu.VMEM((_C, dh), jnp.float32),
          pltpu.VMEM((_C, _AUG), jnp.float32),
          pltpu.VMEM_SHARED((npad, dh), jnp.float32),
          pltpu.VMEM_SHARED((npad, _AUG), jnp.float32),
          pltpu.SemaphoreType.DMA,
          pltpu.SemaphoreType.DMA,
          pltpu.SemaphoreType.DMA,
          pltpu.SemaphoreType.DMA,
      ],
      compiler_params=pltpu.CompilerParams(use_tc_tiling_on_sc=False),
  )
  def k(srcx_h, dst_h, ef_h, feat2_h, psum_h, paug_h,
        sidx, didx, r0b, r1b, a0b, a1b, zb, zab, acc, eacc,
        g0, g1, e0, e1):
    cid = lax.axis_index("c")
    sid = lax.axis_index("s")
    base = sid * ept

    pltpu.sync_copy(srcx_h.at[cid, pl.ds(base, ept)], sidx)
    pltpu.sync_copy(dst_h.at[pl.ds(base, ept)], didx)

    def src_at(ci):
      return sidx.at[pl.ds(ci * _C, _C)]

    def dst_at(ci):
      return didx.at[pl.ds(ci * _C, _C)]

    pltpu.async_copy(feat2_h.at[src_at(0)], r0b, g0)
    pltpu.async_copy(feat2_h.at[src_at(1)], r1b, g1)

    zeros16 = jnp.zeros((16,), jnp.float32)
    onecol = jnp.where(lax.iota(jnp.int32, 16) == 0,
                       jnp.float32(1.0), jnp.float32(0.0))

    def zrow(i, carry):
      for j in range(dh // 16):
        zb[i, pl.ds(j * 16, 16)] = zeros16
      for j in range(_AUG // 16):
        zab[i, pl.ds(j * 16, 16)] = zeros16
      a0b[i, pl.ds(de, 16)] = onecol
      a1b[i, pl.ds(de, 16)] = onecol
      return carry
    lax.fori_loop(0, _C, zrow, 0)

    rr0 = sid * rpt
    for kk in range(rpt // _C):
      pltpu.sync_copy(zb, acc.at[pl.ds(rr0 + kk * _C, _C)])
      pltpu.sync_copy(zab, eacc.at[pl.ds(rr0 + kk * _C, _C)])
    plsc.subcore_barrier()

    eoff = cid * nef

    def ef_at(ei):
      return ef_h.at[pl.ds(base + ei * _C, _C)]

    pltpu.async_copy(ef_at(eoff), a0b.at[:, pl.ds(0, de)], e0)
    pltpu.async_copy(ef_at(eoff + 1), a1b.at[:, pl.ds(0, de)], e1)

    def gwait(buf, sem, ci):
      pltpu.make_async_copy(feat2_h.at[src_at(ci)], buf, sem).wait()

    def ewait(buf, sem, ei):
      pltpu.make_async_copy(ef_at(ei), buf.at[:, pl.ds(0, de)], sem).wait()

    def body(g, carry):
      for (buf, sem, off) in ((r0b, g0, 0), (r1b, g1, 1),
                              (r0b, g0, 2), (r1b, g1, 3)):
        ci = 4 * g + off
        gwait(buf, sem, ci)
        pltpu.sync_copy(buf, acc.at[dst_at(ci)], add=True)
        pltpu.async_copy(feat2_h.at[src_at(ci + 2)], buf, sem)
      for (buf, sem, off) in ((a0b, e0, 0), (a1b, e1, 1)):
        ei = eoff + 2 * g + off
        ewait(buf, sem, ei)
        pltpu.sync_copy(buf, eacc.at[dst_at(ei)], add=True)
        nxt = 2 * g + off + 2

        @pl.when(nxt < nef)
        def _():
          pltpu.async_copy(ef_at(eoff + nxt), buf.at[:, pl.ds(0, de)], sem)
      return carry

    niter = (nch - 2) // 4
    lax.fori_loop(0, niter, body, 0)

    gwait(r0b, g0, nch - 2)
    pltpu.sync_copy(r0b, acc.at[dst_at(nch - 2)], add=True)
    gwait(r1b, g1, nch - 1)
    pltpu.sync_copy(r1b, acc.at[dst_at(nch - 1)], add=True)
    if nef % 2:
      ewait(a0b, e0, eoff + nef - 1)
      pltpu.sync_copy(a0b, eacc.at[dst_at(eoff + nef - 1)], add=True)

    plsc.subcore_barrier()
    pltpu.sync_copy(acc.at[pl.ds(rr0, rpt)],
                    psum_h.at[cid, pl.ds(rr0, rpt)])
    pltpu.sync_copy(eacc.at[pl.ds(rr0, rpt)],
                    paug_h.at[cid, pl.ds(rr0, rpt)])

  return k(srcx, dst, ef, feat2)


def _combine(feat, psum, paug, w_self, w_neigh, bias2d):
  n, d = feat.shape
  dh = psum.shape[-1]
  dk = w_neigh.shape[0]
  de = dk - d
  blk = 1000
  grid = (n // blk,)

  def body(f_ref, ps_ref, pa_ref, ws_ref, wn_ref, b_ref, o_ref):
    a = pa_ref[0] + pa_ref[1]
    e = a[:, 0:de]
    dg = a[:, de:de + 1]
    scale = 1.0 / jnp.maximum(dg, 1.0)
    wn = wn_ref[...]
    h = (jnp.dot(ps_ref[0], wn[0:dh], preferred_element_type=jnp.float32)
         + jnp.dot(ps_ref[1], wn[dh:2 * dh],
                   preferred_element_type=jnp.float32)
         + jnp.dot(e, wn[2 * dh:dk], preferred_element_type=jnp.float32))
    o_ref[...] = (jnp.dot(f_ref[...], ws_ref[...],
                          preferred_element_type=jnp.float32)
                  + h * scale + b_ref[...])

  return pl.pallas_call(
      body,
      grid=grid,
      in_specs=[
          pl.BlockSpec((blk, d), lambda i: (i, 0)),
          pl.BlockSpec((_NC, blk, dh), lambda i: (0, i, 0)),
          pl.BlockSpec((_NC, blk, _AUG), lambda i: (0, i, 0)),
          pl.BlockSpec((d, d), lambda i: (0, 0)),
          pl.BlockSpec((dk, d), lambda i: (0, 0)),
          pl.BlockSpec((1, d), lambda i: (0, 0)),
      ],
      out_specs=pl.BlockSpec((blk, d), lambda i: (i, 0)),
      out_shape=jax.ShapeDtypeStruct((n, d), jnp.float32),
  )(feat, psum, paug, w_self, w_neigh, bias2d)


def kernel(feat, edge_index, edge_feats, W_self, W_neigh, bias):
  n, d = feat.shape
  dh = d // _NC
  src = edge_index[0]
  dst = edge_index[1]
  feat2 = feat.reshape(_NC * n, dh)
  srcx = jnp.stack([2 * src, 2 * src + 1])
  psum, paug = _sc_segment_sums(feat2, srcx, dst, edge_feats, n)
  return _combine(feat, psum, paug, W_self, W_neigh, bias.reshape(1, d))
